# Initial kernel scaffold; baseline (speedup 1.0000x reference)
import jax
import jax.numpy as jnp
from jax.experimental import pallas as pl


def kernel(x, edge_index, edge_attr, batch, W1_rel, b1, W1_root, W2_rel, b2, W2_root, W3_rel, b3, W3_root, W4_rel, b4, W4_root):
    raise NotImplementedError("write your pallas kernel here")


# speedup vs baseline: 18.6102x; 18.6102x over previous
import functools

import jax
import jax.numpy as jnp
from jax import lax
from jax.experimental import pallas as pl
from jax.experimental.pallas import tpu as pltpu
from jax.experimental.pallas import tpu_sc as plsc

_N = 10000
_NPAD = 10240
_E = 320000
_D = 128
_H = 1000
_HP = 1024
_G = 64

_NCORE = 2
_NSUB = 16
_NW = _NCORE * _NSUB
_EPW = _E // _NW
_CH = 80
_NCH = _EPW // _CH
_RPS = _NPAD // _NSUB

_BN = 1024
_BN2 = 2048


def _sc_scatter(src3, dst3, feats):
    mesh = plsc.VectorSubcoreMesh(core_axis_name="c", subcore_axis_name="s")

    @functools.partial(
        pl.kernel,
        mesh=mesh,
        out_type=jax.ShapeDtypeStruct((_NCORE, _NPAD, _D), jnp.float32),
        scratch_types=[
            pltpu.VMEM_SHARED((_NPAD, _D), jnp.float32),
            pltpu.VMEM((_EPW,), jnp.int32),
            pltpu.VMEM((_NCH, _CH), jnp.int32),
            pltpu.VMEM((_CH, _D), jnp.float32),
            pltpu.VMEM((_CH, _D), jnp.float32),
            pltpu.SemaphoreType.DMA,
            pltpu.SemaphoreType.DMA,
        ],
    )
    def k(src_hbm, dst_hbm, x_hbm, out_hbm, acc, sidx, didx, buf0, buf1,
          sem0, sem1):
        c = lax.axis_index("c")
        s = lax.axis_index("s")
        wid = s * _NCORE + c

        zv = jnp.zeros((16,), jnp.float32)

        def _zrow(i, carry):
            for j in range(_D // 16):
                buf0[i, pl.ds(j * 16, 16)] = zv
            return carry

        lax.fori_loop(0, _CH, _zrow, 0)
        for t in range(_RPS // _CH):
            pltpu.sync_copy(buf0, acc.at[pl.ds(s * _RPS + t * _CH, _CH)])
        plsc.subcore_barrier()

        pltpu.sync_copy(src_hbm.at[wid], sidx)
        pltpu.sync_copy(dst_hbm.at[wid], didx)

        def gather(chunk, buf, sem):
            pltpu.async_copy(x_hbm.at[sidx.at[pl.ds(chunk * _CH, _CH)]], buf, sem)

        def gwait(buf, sem):
            pltpu.make_async_copy(x_hbm.at[sidx.at[pl.ds(0, _CH)]], buf, sem).wait()

        def scat(chunk, buf):
            pltpu.sync_copy(buf, acc.at[didx.at[chunk]], add=True)

        gather(0, buf0, sem0)

        def step(j, carry):
            c0 = 2 * j
            gwait(buf0, sem0)
            gather(c0 + 1, buf1, sem1)
            scat(c0, buf0)
            gwait(buf1, sem1)
            gather(c0 + 2, buf0, sem0)
            scat(c0 + 1, buf1)
            return carry

        lax.fori_loop(0, (_NCH - 1) // 2, step, 0)
        gwait(buf0, sem0)
        scat(_NCH - 1, buf0)

        plsc.subcore_barrier()
        pltpu.sync_copy(acc.at[pl.ds(s * _RPS, _RPS)],
                        out_hbm.at[c, pl.ds(s * _RPS, _RPS)])

    return k(src3, dst3, feats)


def _conv_body(p0_ref, p1_ref, x_ref, wr_ref, wb_ref, b_ref, o_ref):
    agg = p0_ref[...] + p1_ref[...]
    acc = jnp.dot(agg, wr_ref[...], preferred_element_type=jnp.float32)
    acc = acc + jnp.dot(x_ref[...], wb_ref[...], preferred_element_type=jnp.float32)
    o_ref[...] = jnp.maximum(acc + b_ref[...], 0.0)


def _conv(p0, p1, x, wr, wb, b):
    return pl.pallas_call(
        _conv_body,
        grid=(_NPAD // _BN,),
        in_specs=[
            pl.BlockSpec((_BN, _D), lambda i: (i, 0)),
            pl.BlockSpec((_BN, _D), lambda i: (i, 0)),
            pl.BlockSpec((_BN, _D), lambda i: (i, 0)),
            pl.BlockSpec((_D, _HP), lambda i: (0, 0)),
            pl.BlockSpec((_D, _HP), lambda i: (0, 0)),
            pl.BlockSpec((1, _HP), lambda i: (0, 0)),
        ],
        out_specs=pl.BlockSpec((_BN, _HP), lambda i: (i, 0)),
        out_shape=jax.ShapeDtypeStruct((_NPAD, _HP), jnp.float32),
    )(p0, p1, x, wr, wb, b)


def _mm2_body(h_ref, wa_ref, wb_ref, a_ref, b_ref):
    h = h_ref[...]
    a_ref[...] = jnp.dot(h, wa_ref[...], preferred_element_type=jnp.float32)
    b_ref[...] = jnp.dot(h, wb_ref[...], preferred_element_type=jnp.float32)


def _mm2(h, wa, wb):
    return pl.pallas_call(
        _mm2_body,
        grid=(_NPAD // _BN,),
        in_specs=[
            pl.BlockSpec((_BN, _HP), lambda i: (i, 0)),
            pl.BlockSpec((_HP, _D), lambda i: (0, 0)),
            pl.BlockSpec((_HP, _D), lambda i: (0, 0)),
        ],
        out_specs=[
            pl.BlockSpec((_BN, _D), lambda i: (i, 0)),
            pl.BlockSpec((_BN, _D), lambda i: (i, 0)),
        ],
        out_shape=[
            jax.ShapeDtypeStruct((_NPAD, _D), jnp.float32),
            jax.ShapeDtypeStruct((_NPAD, _D), jnp.float32),
        ],
    )(h, wa, wb)


def _ew2_body(p0_ref, p1_ref, r_ref, b_ref, bid_ref, h_ref, sums_ref, cnts_ref):
    i = pl.program_id(0)
    h = jnp.maximum(p0_ref[...] + p1_ref[...] + r_ref[...] + b_ref[...], 0.0)
    h_ref[...] = h
    bid = bid_ref[0]
    iota = lax.broadcasted_iota(jnp.int32, (_G, 1), 0)
    onehot = (bid == iota).astype(jnp.float32)

    @pl.when(i == 0)
    def _():
        sums_ref[...] = jnp.zeros_like(sums_ref)
        cnts_ref[...] = jnp.zeros_like(cnts_ref)

    sums_ref[...] += jnp.dot(onehot, h, preferred_element_type=jnp.float32)
    cnts_ref[...] += jnp.broadcast_to(
        jnp.sum(onehot, axis=1, keepdims=True), (_G, _D))


def _ew2(p0, p1, r, b, bid3):
    return pl.pallas_call(
        _ew2_body,
        grid=(_NPAD // _BN2,),
        in_specs=[
            pl.BlockSpec((_BN2, _D), lambda i: (i, 0)),
            pl.BlockSpec((_BN2, _D), lambda i: (i, 0)),
            pl.BlockSpec((_BN2, _D), lambda i: (i, 0)),
            pl.BlockSpec((1, _D), lambda i: (0, 0)),
            pl.BlockSpec((1, 1, _BN2), lambda i: (i, 0, 0)),
        ],
        out_specs=[
            pl.BlockSpec((_BN2, _D), lambda i: (i, 0)),
            pl.BlockSpec((_G, _D), lambda i: (0, 0)),
            pl.BlockSpec((_G, _D), lambda i: (0, 0)),
        ],
        out_shape=[
            jax.ShapeDtypeStruct((_NPAD, _D), jnp.float32),
            jax.ShapeDtypeStruct((_G, _D), jnp.float32),
            jax.ShapeDtypeStruct((_G, _D), jnp.float32),
        ],
    )(p0, p1, r, b, bid3)


def _out_body(p0_ref, p1_ref, r_ref, b_ref, sums_ref, cnts_ref, o_ref, enc_ref):
    i = pl.program_id(0)
    o_ref[...] = p0_ref[...] + p1_ref[...] + r_ref[...] + b_ref[...]

    @pl.when(i == 0)
    def _():
        enc_ref[...] = sums_ref[...] / jnp.maximum(cnts_ref[...], 1.0)


def _out(p0, p1, r, b, sums, cnts):
    return pl.pallas_call(
        _out_body,
        grid=(_NPAD // _BN2,),
        in_specs=[
            pl.BlockSpec((_BN2, _D), lambda i: (i, 0)),
            pl.BlockSpec((_BN2, _D), lambda i: (i, 0)),
            pl.BlockSpec((_BN2, _D), lambda i: (i, 0)),
            pl.BlockSpec((1, _D), lambda i: (0, 0)),
            pl.BlockSpec((_G, _D), lambda i: (0, 0)),
            pl.BlockSpec((_G, _D), lambda i: (0, 0)),
        ],
        out_specs=[
            pl.BlockSpec((_BN2, _D), lambda i: (i, 0)),
            pl.BlockSpec((_G, _D), lambda i: (0, 0)),
        ],
        out_shape=[
            jax.ShapeDtypeStruct((_NPAD, _D), jnp.float32),
            jax.ShapeDtypeStruct((_G, _D), jnp.float32),
        ],
    )(p0, p1, r, b, sums, cnts)


def kernel(x, edge_index, edge_attr, batch,
           W1_rel, b1, W1_root, W2_rel, b2, W2_root,
           W3_rel, b3, W3_root, W4_rel, b4, W4_root):
    del edge_attr
    xp = jnp.pad(x, ((0, _NPAD - _N), (0, 0)))
    src2 = edge_index[0].reshape(_NW, _EPW)
    dst3 = edge_index[1].reshape(_NW, _NCH, _CH)
    bid3 = jnp.pad(batch, (0, _NPAD - _N), constant_values=_G).reshape(
        _NPAD // _BN2, 1, _BN2)

    w1r = jnp.pad(W1_rel, ((0, 0), (0, _HP - _H)))
    w1b = jnp.pad(W1_root, ((0, 0), (0, _HP - _H)))
    b1p = jnp.pad(b1, (0, _HP - _H)).reshape(1, _HP)
    w2a = jnp.pad(W2_rel, ((0, _HP - _H), (0, 0)))
    w2b = jnp.pad(W2_root, ((0, _HP - _H), (0, 0)))
    b2p = b2.reshape(1, _D)
    w3r = jnp.pad(W3_rel, ((0, 0), (0, _HP - _H)))
    w3b = jnp.pad(W3_root, ((0, 0), (0, _HP - _H)))
    b3p = jnp.pad(b3, (0, _HP - _H)).reshape(1, _HP)
    w4a = jnp.pad(W4_rel, ((0, _HP - _H), (0, 0)))
    w4b = jnp.pad(W4_root, ((0, _HP - _H), (0, 0)))
    b4p = b4.reshape(1, _D)

    pa = _sc_scatter(src2, dst3, xp)
    h1 = _conv(pa[0], pa[1], xp, w1r, w1b, b1p)
    m2, r2 = _mm2(h1, w2a, w2b)
    pb = _sc_scatter(src2, dst3, m2)
    h2, sums, cnts = _ew2(pb[0], pb[1], r2, b2p, bid3)
    pc = _sc_scatter(src2, dst3, h2)
    h3 = _conv(pc[0], pc[1], h2, w3r, w3b, b3p)
    m4, r4 = _mm2(h3, w4a, w4b)
    pd = _sc_scatter(src2, dst3, m4)
    out_full, encoded = _out(pd[0], pd[1], r4, b4p, sums, cnts)
    return (out_full[:_N], encoded)


# speedup vs baseline: 19.6971x; 1.0584x over previous
import functools

import jax
import jax.numpy as jnp
from jax import lax
from jax.experimental import pallas as pl
from jax.experimental.pallas import tpu as pltpu
from jax.experimental.pallas import tpu_sc as plsc

_N = 10000
_NPAD = 10240
_E = 320000
_D = 128
_H = 1000
_HP = 1024
_G = 64

_NCORE = 2
_NSUB = 16
_NW = _NCORE * _NSUB
_EPW = _E // _NW
_CH = 80
_NCH = _EPW // _CH
_RPS = _NPAD // _NSUB

_BN = 1024
_BN2 = 2048


def _sc_scatter(src3, dst3, feats):
    mesh = plsc.VectorSubcoreMesh(core_axis_name="c", subcore_axis_name="s")

    @functools.partial(
        pl.kernel,
        mesh=mesh,
        out_type=jax.ShapeDtypeStruct((_NCORE, _NPAD, _D), jnp.float32),
        scratch_types=[
            pltpu.VMEM_SHARED((_NPAD, _D), jnp.float32),
            pltpu.VMEM((_EPW,), jnp.int32),
            pltpu.VMEM((_NCH, _CH), jnp.int32),
            pltpu.VMEM((_CH, _D), jnp.float32),
            pltpu.VMEM((_CH, _D), jnp.float32),
            pltpu.SemaphoreType.DMA,
            pltpu.SemaphoreType.DMA,
            pltpu.SemaphoreType.DMA,
            pltpu.SemaphoreType.DMA,
        ],
    )
    def k(src_hbm, dst_hbm, x_hbm, out_hbm, acc, sidx, didx, buf0, buf1,
          sem0, sem1, ssem0, ssem1):
        c = lax.axis_index("c")
        s = lax.axis_index("s")
        wid = s * _NCORE + c

        zv = jnp.zeros((16,), jnp.float32)

        def _zrow(i, carry):
            for j in range(_D // 16):
                buf0[i, pl.ds(j * 16, 16)] = zv
            return carry

        lax.fori_loop(0, _CH, _zrow, 0)
        for t in range(_RPS // _CH):
            pltpu.sync_copy(buf0, acc.at[pl.ds(s * _RPS + t * _CH, _CH)])
        plsc.subcore_barrier()

        pltpu.sync_copy(src_hbm.at[wid], sidx)
        pltpu.sync_copy(dst_hbm.at[wid], didx)

        def gather(chunk, buf, sem):
            pltpu.async_copy(x_hbm.at[sidx.at[pl.ds(chunk * _CH, _CH)]], buf, sem)

        def gwait(buf, sem):
            pltpu.make_async_copy(x_hbm.at[sidx.at[pl.ds(0, _CH)]], buf, sem).wait()

        gather(0, buf0, sem0)
        gather(1, buf1, sem1)

        def step(j, carry):
            a = 2 * j
            gwait(buf0, sem0)
            d0 = pltpu.async_copy(buf0, acc.at[didx.at[a]], ssem0, add=True)
            gwait(buf1, sem1)
            d1 = pltpu.async_copy(buf1, acc.at[didx.at[a + 1]], ssem1, add=True)
            d0.wait()

            @pl.when(a + 2 < _NCH)
            def _():
                gather(a + 2, buf0, sem0)

            d1.wait()

            @pl.when(a + 3 < _NCH)
            def _():
                gather(a + 3, buf1, sem1)

            return carry

        lax.fori_loop(0, _NCH // 2, step, 0)
        gwait(buf0, sem0)
        pltpu.sync_copy(buf0, acc.at[didx.at[_NCH - 1]], add=True)

        plsc.subcore_barrier()
        pltpu.sync_copy(acc.at[pl.ds(s * _RPS, _RPS)],
                        out_hbm.at[c, pl.ds(s * _RPS, _RPS)])

    return k(src3, dst3, feats)


def _layer_body(p0_ref, p1_ref, x_ref, wr_ref, wb_ref, b_ref, wa_ref, wc_ref,
                m_ref, r_ref):
    agg = p0_ref[...] + p1_ref[...]
    acc = jnp.dot(agg, wr_ref[...], preferred_element_type=jnp.float32)
    acc = acc + jnp.dot(x_ref[...], wb_ref[...], preferred_element_type=jnp.float32)
    h = jnp.maximum(acc + b_ref[...], 0.0)
    m_ref[...] = jnp.dot(h, wa_ref[...], preferred_element_type=jnp.float32)
    r_ref[...] = jnp.dot(h, wc_ref[...], preferred_element_type=jnp.float32)


def _layer(p0, p1, x, wr, wb, b, wa, wc):
    return pl.pallas_call(
        _layer_body,
        grid=(_NPAD // _BN,),
        in_specs=[
            pl.BlockSpec((_BN, _D), lambda i: (i, 0)),
            pl.BlockSpec((_BN, _D), lambda i: (i, 0)),
            pl.BlockSpec((_BN, _D), lambda i: (i, 0)),
            pl.BlockSpec((_D, _HP), lambda i: (0, 0)),
            pl.BlockSpec((_D, _HP), lambda i: (0, 0)),
            pl.BlockSpec((1, _HP), lambda i: (0, 0)),
            pl.BlockSpec((_HP, _D), lambda i: (0, 0)),
            pl.BlockSpec((_HP, _D), lambda i: (0, 0)),
        ],
        out_specs=[
            pl.BlockSpec((_BN, _D), lambda i: (i, 0)),
            pl.BlockSpec((_BN, _D), lambda i: (i, 0)),
        ],
        out_shape=[
            jax.ShapeDtypeStruct((_NPAD, _D), jnp.float32),
            jax.ShapeDtypeStruct((_NPAD, _D), jnp.float32),
        ],
    )(p0, p1, x, wr, wb, b, wa, wc)


def _ew2_body(p0_ref, p1_ref, r_ref, b_ref, bid_ref, h_ref, sums_ref, cnts_ref):
    i = pl.program_id(0)
    h = jnp.maximum(p0_ref[...] + p1_ref[...] + r_ref[...] + b_ref[...], 0.0)
    h_ref[...] = h
    bid = bid_ref[0]
    iota = lax.broadcasted_iota(jnp.int32, (_G, 1), 0)
    onehot = (bid == iota).astype(jnp.float32)

    @pl.when(i == 0)
    def _():
        sums_ref[...] = jnp.zeros_like(sums_ref)
        cnts_ref[...] = jnp.zeros_like(cnts_ref)

    sums_ref[...] += jnp.dot(onehot, h, preferred_element_type=jnp.float32)
    cnts_ref[...] += jnp.broadcast_to(
        jnp.sum(onehot, axis=1, keepdims=True), (_G, _D))


def _ew2(p0, p1, r, b, bid3):
    return pl.pallas_call(
        _ew2_body,
        grid=(_NPAD // _BN2,),
        in_specs=[
            pl.BlockSpec((_BN2, _D), lambda i: (i, 0)),
            pl.BlockSpec((_BN2, _D), lambda i: (i, 0)),
            pl.BlockSpec((_BN2, _D), lambda i: (i, 0)),
            pl.BlockSpec((1, _D), lambda i: (0, 0)),
            pl.BlockSpec((1, 1, _BN2), lambda i: (i, 0, 0)),
        ],
        out_specs=[
            pl.BlockSpec((_BN2, _D), lambda i: (i, 0)),
            pl.BlockSpec((_G, _D), lambda i: (0, 0)),
            pl.BlockSpec((_G, _D), lambda i: (0, 0)),
        ],
        out_shape=[
            jax.ShapeDtypeStruct((_NPAD, _D), jnp.float32),
            jax.ShapeDtypeStruct((_G, _D), jnp.float32),
            jax.ShapeDtypeStruct((_G, _D), jnp.float32),
        ],
    )(p0, p1, r, b, bid3)


def _out_body(p0_ref, p1_ref, r_ref, b_ref, sums_ref, cnts_ref, o_ref, enc_ref):
    i = pl.program_id(0)
    o_ref[...] = p0_ref[...] + p1_ref[...] + r_ref[...] + b_ref[...]

    @pl.when(i == 0)
    def _():
        enc_ref[...] = sums_ref[...] / jnp.maximum(cnts_ref[...], 1.0)


def _out(p0, p1, r, b, sums, cnts):
    return pl.pallas_call(
        _out_body,
        grid=(_NPAD // _BN2,),
        in_specs=[
            pl.BlockSpec((_BN2, _D), lambda i: (i, 0)),
            pl.BlockSpec((_BN2, _D), lambda i: (i, 0)),
            pl.BlockSpec((_BN2, _D), lambda i: (i, 0)),
            pl.BlockSpec((1, _D), lambda i: (0, 0)),
            pl.BlockSpec((_G, _D), lambda i: (0, 0)),
            pl.BlockSpec((_G, _D), lambda i: (0, 0)),
        ],
        out_specs=[
            pl.BlockSpec((_BN2, _D), lambda i: (i, 0)),
            pl.BlockSpec((_G, _D), lambda i: (0, 0)),
        ],
        out_shape=[
            jax.ShapeDtypeStruct((_NPAD, _D), jnp.float32),
            jax.ShapeDtypeStruct((_G, _D), jnp.float32),
        ],
    )(p0, p1, r, b, sums, cnts)


def kernel(x, edge_index, edge_attr, batch,
           W1_rel, b1, W1_root, W2_rel, b2, W2_root,
           W3_rel, b3, W3_root, W4_rel, b4, W4_root):
    del edge_attr
    xp = jnp.pad(x, ((0, _NPAD - _N), (0, 0)))
    src2 = edge_index[0].reshape(_NW, _EPW)
    dst3 = edge_index[1].reshape(_NW, _NCH, _CH)
    bid3 = jnp.pad(batch, (0, _NPAD - _N), constant_values=_G).reshape(
        _NPAD // _BN2, 1, _BN2)

    w1r = jnp.pad(W1_rel, ((0, 0), (0, _HP - _H)))
    w1b = jnp.pad(W1_root, ((0, 0), (0, _HP - _H)))
    b1p = jnp.pad(b1, (0, _HP - _H)).reshape(1, _HP)
    w2a = jnp.pad(W2_rel, ((0, _HP - _H), (0, 0)))
    w2b = jnp.pad(W2_root, ((0, _HP - _H), (0, 0)))
    b2p = b2.reshape(1, _D)
    w3r = jnp.pad(W3_rel, ((0, 0), (0, _HP - _H)))
    w3b = jnp.pad(W3_root, ((0, 0), (0, _HP - _H)))
    b3p = jnp.pad(b3, (0, _HP - _H)).reshape(1, _HP)
    w4a = jnp.pad(W4_rel, ((0, _HP - _H), (0, 0)))
    w4b = jnp.pad(W4_root, ((0, _HP - _H), (0, 0)))
    b4p = b4.reshape(1, _D)

    pa = _sc_scatter(src2, dst3, xp)
    m2, r2 = _layer(pa[0], pa[1], xp, w1r, w1b, b1p, w2a, w2b)
    pb = _sc_scatter(src2, dst3, m2)
    h2, sums, cnts = _ew2(pb[0], pb[1], r2, b2p, bid3)
    pc = _sc_scatter(src2, dst3, h2)
    m4, r4 = _layer(pc[0], pc[1], h2, w3r, w3b, b3p, w4a, w4b)
    pd = _sc_scatter(src2, dst3, m4)
    out_full, encoded = _out(pd[0], pd[1], r4, b4p, sums, cnts)
    return (out_full[:_N], encoded)


# speedup vs baseline: 24.0281x; 1.2199x over previous
import functools

import jax
import jax.numpy as jnp
from jax import lax
from jax.experimental import pallas as pl
from jax.experimental.pallas import tpu as pltpu
from jax.experimental.pallas import tpu_sc as plsc

_N = 10000
_NPAD = 10240
_E = 320000
_D = 128
_H = 1000
_HP = 1024
_G = 64

_NCORE = 2
_NSUB = 16
_NW = _NCORE * _NSUB
_EPW = _E // _NW
_CH = 80
_NCH = _EPW // _CH
_RPS = _NPAD // _NSUB

_BN = 1024
_BN2 = 2048


def _sc_scatter(packed2, feats):
    mesh = plsc.VectorSubcoreMesh(core_axis_name="c", subcore_axis_name="s")

    @functools.partial(
        pl.kernel,
        mesh=mesh,
        out_type=jax.ShapeDtypeStruct((_NCORE, _NPAD, _D), jnp.float32),
        scratch_types=[
            pltpu.VMEM_SHARED((_NPAD, _D), jnp.float32),
            pltpu.VMEM((_EPW,), jnp.int32),
            pltpu.VMEM((3, _CH), jnp.int32),
            pltpu.VMEM((3, _CH), jnp.int32),
            pltpu.VMEM((_CH, _D), jnp.float32),
            pltpu.VMEM((_CH, _D), jnp.float32),
            pltpu.VMEM((_CH, _D), jnp.float32),
            pltpu.SemaphoreType.DMA,
            pltpu.SemaphoreType.DMA,
            pltpu.SemaphoreType.DMA,
            pltpu.SemaphoreType.DMA,
            pltpu.SemaphoreType.DMA,
            pltpu.SemaphoreType.DMA,
        ],
    )
    def k(pk_hbm, x_hbm, out_hbm, acc, pkv, sidx, didx, buf0, buf1, buf2,
          gs0, gs1, gs2, ss0, ss1, ss2):
        c = lax.axis_index("c")
        s = lax.axis_index("s")
        wid = s * _NCORE + c
        bufs = (buf0, buf1, buf2)
        gsems = (gs0, gs1, gs2)
        ssems = (ss0, ss1, ss2)

        zv = jnp.zeros((16,), jnp.float32)

        def _zrow(i, carry):
            for j in range(_D // 16):
                buf0[i, pl.ds(j * 16, 16)] = zv
            return carry

        lax.fori_loop(0, _CH, _zrow, 0)
        for t in range(_RPS // _CH):
            pltpu.sync_copy(buf0, acc.at[pl.ds(s * _RPS + t * _CH, _CH)])
        plsc.subcore_barrier()

        pltpu.sync_copy(pk_hbm.at[wid], pkv)

        def unpack(chunk, b):
            for r in range(_CH // 16):
                v = pkv[pl.ds(chunk * _CH + r * 16, 16)]
                sidx[b, pl.ds(r * 16, 16)] = v & 0xFFFF
                didx[b, pl.ds(r * 16, 16)] = v >> 16

        def gather(chunk, b):
            pltpu.async_copy(x_hbm.at[sidx.at[b]], bufs[b], gsems[b])

        def gwait(b):
            pltpu.make_async_copy(x_hbm.at[sidx.at[b]], bufs[b], gsems[b]).wait()

        def scat(b):
            return pltpu.async_copy(bufs[b], acc.at[didx.at[b]], ssems[b],
                                    add=True)

        for b in range(3):
            unpack(b, b)
            gather(b, b)

        def step(g, carry):
            c0 = 3 * g
            ds = []
            for r in range(3):
                gwait(r)
                ds.append(scat(r))
            for r in range(3):
                ds[r].wait()

                @pl.when(c0 + 3 + r < _NCH)
                def _():
                    unpack(c0 + 3 + r, r)
                    gather(c0 + 3 + r, r)

            return carry

        lax.fori_loop(0, _NCH // 3, step, 0)
        gwait(0)
        d0 = scat(0)
        gwait(1)
        d1 = scat(1)
        d0.wait()
        d1.wait()

        plsc.subcore_barrier()
        pltpu.sync_copy(acc.at[pl.ds(s * _RPS, _RPS)],
                        out_hbm.at[c, pl.ds(s * _RPS, _RPS)])

    return k(packed2, feats)


def _layer_body(p0_ref, p1_ref, x_ref, wr_ref, wb_ref, b_ref, wa_ref, wc_ref,
                m_ref, r_ref):
    agg = p0_ref[...] + p1_ref[...]
    acc = jnp.dot(agg, wr_ref[...], preferred_element_type=jnp.float32)
    acc = acc + jnp.dot(x_ref[...], wb_ref[...], preferred_element_type=jnp.float32)
    h = jnp.maximum(acc + b_ref[...], 0.0)
    m_ref[...] = jnp.dot(h, wa_ref[...], preferred_element_type=jnp.float32)
    r_ref[...] = jnp.dot(h, wc_ref[...], preferred_element_type=jnp.float32)


def _layer(p0, p1, x, wr, wb, b, wa, wc):
    return pl.pallas_call(
        _layer_body,
        grid=(_NPAD // _BN,),
        in_specs=[
            pl.BlockSpec((_BN, _D), lambda i: (i, 0)),
            pl.BlockSpec((_BN, _D), lambda i: (i, 0)),
            pl.BlockSpec((_BN, _D), lambda i: (i, 0)),
            pl.BlockSpec((_D, _HP), lambda i: (0, 0)),
            pl.BlockSpec((_D, _HP), lambda i: (0, 0)),
            pl.BlockSpec((1, _HP), lambda i: (0, 0)),
            pl.BlockSpec((_HP, _D), lambda i: (0, 0)),
            pl.BlockSpec((_HP, _D), lambda i: (0, 0)),
        ],
        out_specs=[
            pl.BlockSpec((_BN, _D), lambda i: (i, 0)),
            pl.BlockSpec((_BN, _D), lambda i: (i, 0)),
        ],
        out_shape=[
            jax.ShapeDtypeStruct((_NPAD, _D), jnp.float32),
            jax.ShapeDtypeStruct((_NPAD, _D), jnp.float32),
        ],
    )(p0, p1, x, wr, wb, b, wa, wc)


def _ew2_body(p0_ref, p1_ref, r_ref, b_ref, bid_ref, h_ref, sums_ref, cnts_ref):
    i = pl.program_id(0)
    h = jnp.maximum(p0_ref[...] + p1_ref[...] + r_ref[...] + b_ref[...], 0.0)
    h_ref[...] = h
    bid = bid_ref[0]
    iota = lax.broadcasted_iota(jnp.int32, (_G, 1), 0)
    onehot = (bid == iota).astype(jnp.float32)

    @pl.when(i == 0)
    def _():
        sums_ref[...] = jnp.zeros_like(sums_ref)
        cnts_ref[...] = jnp.zeros_like(cnts_ref)

    sums_ref[...] += jnp.dot(onehot, h, preferred_element_type=jnp.float32)
    cnts_ref[...] += jnp.broadcast_to(
        jnp.sum(onehot, axis=1, keepdims=True), (_G, _D))


def _ew2(p0, p1, r, b, bid3):
    return pl.pallas_call(
        _ew2_body,
        grid=(_NPAD // _BN2,),
        in_specs=[
            pl.BlockSpec((_BN2, _D), lambda i: (i, 0)),
            pl.BlockSpec((_BN2, _D), lambda i: (i, 0)),
            pl.BlockSpec((_BN2, _D), lambda i: (i, 0)),
            pl.BlockSpec((1, _D), lambda i: (0, 0)),
            pl.BlockSpec((1, 1, _BN2), lambda i: (i, 0, 0)),
        ],
        out_specs=[
            pl.BlockSpec((_BN2, _D), lambda i: (i, 0)),
            pl.BlockSpec((_G, _D), lambda i: (0, 0)),
            pl.BlockSpec((_G, _D), lambda i: (0, 0)),
        ],
        out_shape=[
            jax.ShapeDtypeStruct((_NPAD, _D), jnp.float32),
            jax.ShapeDtypeStruct((_G, _D), jnp.float32),
            jax.ShapeDtypeStruct((_G, _D), jnp.float32),
        ],
    )(p0, p1, r, b, bid3)


def _out_body(p0_ref, p1_ref, r_ref, b_ref, sums_ref, cnts_ref, o_ref, enc_ref):
    i = pl.program_id(0)
    o_ref[...] = p0_ref[...] + p1_ref[...] + r_ref[...] + b_ref[...]

    @pl.when(i == 0)
    def _():
        enc_ref[...] = sums_ref[...] / jnp.maximum(cnts_ref[...], 1.0)


def _out(p0, p1, r, b, sums, cnts):
    return pl.pallas_call(
        _out_body,
        grid=(_NPAD // _BN2,),
        in_specs=[
            pl.BlockSpec((_BN2, _D), lambda i: (i, 0)),
            pl.BlockSpec((_BN2, _D), lambda i: (i, 0)),
            pl.BlockSpec((_BN2, _D), lambda i: (i, 0)),
            pl.BlockSpec((1, _D), lambda i: (0, 0)),
            pl.BlockSpec((_G, _D), lambda i: (0, 0)),
            pl.BlockSpec((_G, _D), lambda i: (0, 0)),
        ],
        out_specs=[
            pl.BlockSpec((_BN2, _D), lambda i: (i, 0)),
            pl.BlockSpec((_G, _D), lambda i: (0, 0)),
        ],
        out_shape=[
            jax.ShapeDtypeStruct((_NPAD, _D), jnp.float32),
            jax.ShapeDtypeStruct((_G, _D), jnp.float32),
        ],
    )(p0, p1, r, b, sums, cnts)


def kernel(x, edge_index, edge_attr, batch,
           W1_rel, b1, W1_root, W2_rel, b2, W2_root,
           W3_rel, b3, W3_root, W4_rel, b4, W4_root):
    del edge_attr
    xp = jnp.pad(x, ((0, _NPAD - _N), (0, 0)))
    packed2 = ((edge_index[1] << 16) | edge_index[0]).reshape(_NW, _EPW)
    bid3 = jnp.pad(batch, (0, _NPAD - _N), constant_values=_G).reshape(
        _NPAD // _BN2, 1, _BN2)

    w1r = jnp.pad(W1_rel, ((0, 0), (0, _HP - _H)))
    w1b = jnp.pad(W1_root, ((0, 0), (0, _HP - _H)))
    b1p = jnp.pad(b1, (0, _HP - _H)).reshape(1, _HP)
    w2a = jnp.pad(W2_rel, ((0, _HP - _H), (0, 0)))
    w2b = jnp.pad(W2_root, ((0, _HP - _H), (0, 0)))
    b2p = b2.reshape(1, _D)
    w3r = jnp.pad(W3_rel, ((0, 0), (0, _HP - _H)))
    w3b = jnp.pad(W3_root, ((0, 0), (0, _HP - _H)))
    b3p = jnp.pad(b3, (0, _HP - _H)).reshape(1, _HP)
    w4a = jnp.pad(W4_rel, ((0, _HP - _H), (0, 0)))
    w4b = jnp.pad(W4_root, ((0, _HP - _H), (0, 0)))
    b4p = b4.reshape(1, _D)

    pa = _sc_scatter(packed2, xp)
    m2, r2 = _layer(pa[0], pa[1], xp, w1r, w1b, b1p, w2a, w2b)
    pb = _sc_scatter(packed2, m2)
    h2, sums, cnts = _ew2(pb[0], pb[1], r2, b2p, bid3)
    pc = _sc_scatter(packed2, h2)
    m4, r4 = _layer(pc[0], pc[1], h2, w3r, w3b, b3p, w4a, w4b)
    pd = _sc_scatter(packed2, m4)
    out_full, encoded = _out(pd[0], pd[1], r4, b4p, sums, cnts)
    return (out_full[:_N], encoded)


# speedup vs baseline: 24.3317x; 1.0126x over previous
import functools

import jax
import jax.numpy as jnp
from jax import lax
from jax.experimental import pallas as pl
from jax.experimental.pallas import tpu as pltpu
from jax.experimental.pallas import tpu_sc as plsc

_N = 10000
_NPAD = 10240
_E = 320000
_D = 128
_H = 1000
_HP = 1024
_G = 64

_NCORE = 2
_NSUB = 16
_NW = _NCORE * _NSUB
_EPW = _E // _NW
_CH = 80
_NCH = _EPW // _CH
_RPS = _NPAD // _NSUB

_BN = 1024
_BN2 = 2048


def _sc_scatter(packed2, feats):
    mesh = plsc.VectorSubcoreMesh(core_axis_name="c", subcore_axis_name="s")

    @functools.partial(
        pl.kernel,
        mesh=mesh,
        out_type=jax.ShapeDtypeStruct((_NCORE, _NPAD, _D), jnp.float32),
        scratch_types=[
            pltpu.VMEM_SHARED((_NPAD, _D), jnp.float32),
            pltpu.VMEM((_EPW,), jnp.int32),
            pltpu.VMEM((3, _CH), jnp.int32),
            pltpu.VMEM((3, _CH), jnp.int32),
            pltpu.VMEM((_CH, _D), jnp.float32),
            pltpu.VMEM((_CH, _D), jnp.float32),
            pltpu.VMEM((_CH, _D), jnp.float32),
            pltpu.SemaphoreType.DMA,
            pltpu.SemaphoreType.DMA,
            pltpu.SemaphoreType.DMA,
            pltpu.SemaphoreType.DMA,
            pltpu.SemaphoreType.DMA,
            pltpu.SemaphoreType.DMA,
        ],
    )
    def k(pk_hbm, x_hbm, out_hbm, acc, pkv, sidx, didx, buf0, buf1, buf2,
          gs0, gs1, gs2, ss0, ss1, ss2):
        c = lax.axis_index("c")
        s = lax.axis_index("s")
        wid = s * _NCORE + c
        bufs = (buf0, buf1, buf2)
        gsems = (gs0, gs1, gs2)
        ssems = (ss0, ss1, ss2)

        dpk = pltpu.async_copy(pk_hbm.at[wid], pkv, gs2)

        zv = jnp.zeros((16,), jnp.float32)

        def _zrow(i, carry):
            for j in range(_D // 16):
                buf0[i, pl.ds(j * 16, 16)] = zv
            return carry

        lax.fori_loop(0, _CH, _zrow, 0)
        dz = [pltpu.async_copy(buf0, acc.at[pl.ds(s * _RPS + t * _CH, _CH)],
                               ss0)
              for t in range(_RPS // _CH)]
        dpk.wait()

        def unpack(chunk, b):
            for r in range(_CH // 16):
                v = pkv[pl.ds(chunk * _CH + r * 16, 16)]
                sidx[b, pl.ds(r * 16, 16)] = v & 0xFFFF
                didx[b, pl.ds(r * 16, 16)] = v >> 16

        def gather(chunk, b):
            pltpu.async_copy(x_hbm.at[sidx.at[b]], bufs[b], gsems[b])

        def gwait(b):
            pltpu.make_async_copy(x_hbm.at[sidx.at[b]], bufs[b], gsems[b]).wait()

        def scat(b):
            return pltpu.async_copy(bufs[b], acc.at[didx.at[b]], ssems[b],
                                    add=True)

        for b in (1, 2):
            unpack(b, b)
            gather(b, b)
        for d in dz:
            d.wait()
        unpack(0, 0)
        gather(0, 0)
        plsc.subcore_barrier()


        def step(g, carry):
            c0 = 3 * g
            ds = []
            for r in range(3):
                gwait(r)
                ds.append(scat(r))
            for r in range(3):
                ds[r].wait()

                @pl.when(c0 + 3 + r < _NCH)
                def _():
                    unpack(c0 + 3 + r, r)
                    gather(c0 + 3 + r, r)

            return carry

        lax.fori_loop(0, _NCH // 3, step, 0)
        gwait(0)
        d0 = scat(0)
        gwait(1)
        d1 = scat(1)
        d0.wait()
        d1.wait()

        plsc.subcore_barrier()
        pltpu.sync_copy(acc.at[pl.ds(s * _RPS, _RPS)],
                        out_hbm.at[c, pl.ds(s * _RPS, _RPS)])

    return k(packed2, feats)


def _layer_body(p0_ref, p1_ref, x_ref, wr_ref, wb_ref, b_ref, wa_ref, wc_ref,
                m_ref, r_ref):
    agg = p0_ref[...] + p1_ref[...]
    acc = jnp.dot(agg, wr_ref[...], preferred_element_type=jnp.float32)
    acc = acc + jnp.dot(x_ref[...], wb_ref[...], preferred_element_type=jnp.float32)
    h = jnp.maximum(acc + b_ref[...], 0.0)
    m_ref[...] = jnp.dot(h, wa_ref[...], preferred_element_type=jnp.float32)
    r_ref[...] = jnp.dot(h, wc_ref[...], preferred_element_type=jnp.float32)


def _layer(p0, p1, x, wr, wb, b, wa, wc):
    return pl.pallas_call(
        _layer_body,
        grid=(_NPAD // _BN,),
        in_specs=[
            pl.BlockSpec((_BN, _D), lambda i: (i, 0)),
            pl.BlockSpec((_BN, _D), lambda i: (i, 0)),
            pl.BlockSpec((_BN, _D), lambda i: (i, 0)),
            pl.BlockSpec((_D, _HP), lambda i: (0, 0)),
            pl.BlockSpec((_D, _HP), lambda i: (0, 0)),
            pl.BlockSpec((1, _HP), lambda i: (0, 0)),
            pl.BlockSpec((_HP, _D), lambda i: (0, 0)),
            pl.BlockSpec((_HP, _D), lambda i: (0, 0)),
        ],
        out_specs=[
            pl.BlockSpec((_BN, _D), lambda i: (i, 0)),
            pl.BlockSpec((_BN, _D), lambda i: (i, 0)),
        ],
        out_shape=[
            jax.ShapeDtypeStruct((_NPAD, _D), jnp.float32),
            jax.ShapeDtypeStruct((_NPAD, _D), jnp.float32),
        ],
    )(p0, p1, x, wr, wb, b, wa, wc)


def _ew2_body(p0_ref, p1_ref, r_ref, b_ref, bid_ref, h_ref, sums_ref, cnts_ref):
    i = pl.program_id(0)
    h = jnp.maximum(p0_ref[...] + p1_ref[...] + r_ref[...] + b_ref[...], 0.0)
    h_ref[...] = h
    bid = bid_ref[0]
    iota = lax.broadcasted_iota(jnp.int32, (_G, 1), 0)
    onehot = (bid == iota).astype(jnp.float32)

    @pl.when(i == 0)
    def _():
        sums_ref[...] = jnp.zeros_like(sums_ref)
        cnts_ref[...] = jnp.zeros_like(cnts_ref)

    sums_ref[...] += jnp.dot(onehot, h, preferred_element_type=jnp.float32)
    cnts_ref[...] += jnp.broadcast_to(
        jnp.sum(onehot, axis=1, keepdims=True), (_G, _D))


def _ew2(p0, p1, r, b, bid3):
    return pl.pallas_call(
        _ew2_body,
        grid=(_NPAD // _BN2,),
        in_specs=[
            pl.BlockSpec((_BN2, _D), lambda i: (i, 0)),
            pl.BlockSpec((_BN2, _D), lambda i: (i, 0)),
            pl.BlockSpec((_BN2, _D), lambda i: (i, 0)),
            pl.BlockSpec((1, _D), lambda i: (0, 0)),
            pl.BlockSpec((1, 1, _BN2), lambda i: (i, 0, 0)),
        ],
        out_specs=[
            pl.BlockSpec((_BN2, _D), lambda i: (i, 0)),
            pl.BlockSpec((_G, _D), lambda i: (0, 0)),
            pl.BlockSpec((_G, _D), lambda i: (0, 0)),
        ],
        out_shape=[
            jax.ShapeDtypeStruct((_NPAD, _D), jnp.float32),
            jax.ShapeDtypeStruct((_G, _D), jnp.float32),
            jax.ShapeDtypeStruct((_G, _D), jnp.float32),
        ],
    )(p0, p1, r, b, bid3)


def _out_body(p0_ref, p1_ref, r_ref, b_ref, sums_ref, cnts_ref, o_ref, enc_ref):
    i = pl.program_id(0)
    o_ref[...] = p0_ref[...] + p1_ref[...] + r_ref[...] + b_ref[...]

    @pl.when(i == 0)
    def _():
        enc_ref[...] = sums_ref[...] / jnp.maximum(cnts_ref[...], 1.0)


def _out(p0, p1, r, b, sums, cnts):
    return pl.pallas_call(
        _out_body,
        grid=(_NPAD // _BN2,),
        in_specs=[
            pl.BlockSpec((_BN2, _D), lambda i: (i, 0)),
            pl.BlockSpec((_BN2, _D), lambda i: (i, 0)),
            pl.BlockSpec((_BN2, _D), lambda i: (i, 0)),
            pl.BlockSpec((1, _D), lambda i: (0, 0)),
            pl.BlockSpec((_G, _D), lambda i: (0, 0)),
            pl.BlockSpec((_G, _D), lambda i: (0, 0)),
        ],
        out_specs=[
            pl.BlockSpec((_BN2, _D), lambda i: (i, 0)),
            pl.BlockSpec((_G, _D), lambda i: (0, 0)),
        ],
        out_shape=[
            jax.ShapeDtypeStruct((_NPAD, _D), jnp.float32),
            jax.ShapeDtypeStruct((_G, _D), jnp.float32),
        ],
    )(p0, p1, r, b, sums, cnts)


def kernel(x, edge_index, edge_attr, batch,
           W1_rel, b1, W1_root, W2_rel, b2, W2_root,
           W3_rel, b3, W3_root, W4_rel, b4, W4_root):
    del edge_attr
    xp = jnp.pad(x, ((0, _NPAD - _N), (0, 0)))
    packed2 = ((edge_index[1] << 16) | edge_index[0]).reshape(_NW, _EPW)
    bid3 = jnp.pad(batch, (0, _NPAD - _N), constant_values=_G).reshape(
        _NPAD // _BN2, 1, _BN2)

    w1r = jnp.pad(W1_rel, ((0, 0), (0, _HP - _H)))
    w1b = jnp.pad(W1_root, ((0, 0), (0, _HP - _H)))
    b1p = jnp.pad(b1, (0, _HP - _H)).reshape(1, _HP)
    w2a = jnp.pad(W2_rel, ((0, _HP - _H), (0, 0)))
    w2b = jnp.pad(W2_root, ((0, _HP - _H), (0, 0)))
    b2p = b2.reshape(1, _D)
    w3r = jnp.pad(W3_rel, ((0, 0), (0, _HP - _H)))
    w3b = jnp.pad(W3_root, ((0, 0), (0, _HP - _H)))
    b3p = jnp.pad(b3, (0, _HP - _H)).reshape(1, _HP)
    w4a = jnp.pad(W4_rel, ((0, _HP - _H), (0, 0)))
    w4b = jnp.pad(W4_root, ((0, _HP - _H), (0, 0)))
    b4p = b4.reshape(1, _D)

    pa = _sc_scatter(packed2, xp)
    m2, r2 = _layer(pa[0], pa[1], xp, w1r, w1b, b1p, w2a, w2b)
    pb = _sc_scatter(packed2, m2)
    h2, sums, cnts = _ew2(pb[0], pb[1], r2, b2p, bid3)
    pc = _sc_scatter(packed2, h2)
    m4, r4 = _layer(pc[0], pc[1], h2, w3r, w3b, b3p, w4a, w4b)
    pd = _sc_scatter(packed2, m4)
    out_full, encoded = _out(pd[0], pd[1], r4, b4p, sums, cnts)
    return (out_full[:_N], encoded)
